# baseline (speedup 1.0000x reference)
import functools

import jax
import jax.numpy as jnp
from jax import lax
from jax.experimental import pallas as pl
from jax.experimental.pallas import tpu as pltpu
from jax.experimental.pallas import tpu_sc as plsc

K_NN = 16


def _knn_body(xt_ref, ytT_ref, sx_ref, sy_ref, idx_ref, dist_ref):
    M = xt_ref.shape[1]
    N = ytT_ref.shape[2]
    K = idx_ref.shape[1]
    c = jnp.dot(xt_ref[0], ytT_ref[0], preferred_element_type=jnp.float32)
    d = (-2.0 * c + sx_ref[0]) + sy_ref[0]
    di = jax.lax.bitcast_convert_type(d, jnp.int32)
    di = di ^ jax.lax.shift_right_logical(
        jax.lax.shift_right_arithmetic(di, 31), 1)
    col = jax.lax.broadcasted_iota(jnp.int32, (M, N), 1)
    dist_ref[...] = di

    MAXI = jnp.int32(0x7FFFFFFF)
    MINI = jnp.int32(-0x80000000)

    def step(it, t):
        kk = dist_ref[...]
        m = jnp.min(jnp.where(kk > t, kk, MAXI), axis=1, keepdims=True)
        am = jnp.min(jnp.where(kk == m, col, N), axis=1, keepdims=True)
        idx_ref[0, pl.ds(it, 1), :] = am.reshape(1, M)
        return m

    jax.lax.fori_loop(0, K, step, jnp.full((M, 1), MINI, jnp.int32),
                      unroll=False)


def _knn_topk(xt, ytT, sx, sy):
    B, N, _ = xt.shape
    M = 256
    return pl.pallas_call(
        _knn_body,
        grid=(B, N // M),
        in_specs=[
            pl.BlockSpec((1, M, 8), lambda b, i: (b, i, 0)),
            pl.BlockSpec((1, 8, N), lambda b, i: (b, 0, 0)),
            pl.BlockSpec((1, M, 1), lambda b, i: (b, i, 0)),
            pl.BlockSpec((1, 1, N), lambda b, i: (b, 0, 0)),
        ],
        out_specs=pl.BlockSpec((1, K_NN, M), lambda b, i: (b, 0, i)),
        out_shape=jax.ShapeDtypeStruct((B, K_NN, N), jnp.int32),
        scratch_shapes=[pltpu.VMEM((M, N), jnp.int32)],
    )(xt, ytT, sx, sy)


def _sc_gather(table, gidx):
    R, W = table.shape
    P = gidx.shape[0]
    info = plsc.get_sparse_core_info()
    NC, NS = info.num_cores, info.num_subcores
    NW = NC * NS
    C = 128
    per_w = P // NW
    n_chunks = per_w // C
    mesh = plsc.VectorSubcoreMesh(core_axis_name="c", subcore_axis_name="s")

    @functools.partial(
        pl.kernel, mesh=mesh,
        out_type=jax.ShapeDtypeStruct((P, W), jnp.float32),
        scratch_types=[
            pltpu.VMEM((C,), jnp.int32),
            pltpu.VMEM((C, W), jnp.float32),
            pltpu.SemaphoreType.DMA,
        ],
    )
    def k(table_h, gidx_h, out_h, idxv, buf, sem):
        wid = lax.axis_index("s") * NC + lax.axis_index("c")
        base = wid * per_w

        def body(j, _):
            off = base + j * C
            pltpu.sync_copy(gidx_h.at[pl.ds(off, C)], idxv)
            pltpu.async_copy(table_h.at[idxv], buf, sem).wait()
            pltpu.sync_copy(buf, out_h.at[pl.ds(off, C)])
            return 0

        lax.fori_loop(0, n_chunks, body, 0)

    return k(table, gidx)


def _attn_body(qA_ref, aP_ref, g_ref,
               Wp2_ref, bp2_ref, W2A_ref, b2A_ref,
               Wa2_ref, ba2_ref, out_ref):
    M = qA_ref.shape[1]
    K = g_ref.shape[2]
    D = qA_ref.shape[2]

    g = g_ref[0].reshape(M * K, 3 * D)
    aP = aP_ref[0]
    aP_rep = jnp.broadcast_to(aP.reshape(M, 1, D), (M, K, D)).reshape(M * K, D)
    h = jnp.maximum(aP_rep - g[:, 2 * D:3 * D], 0.0)

    delta = jnp.dot(h, Wp2_ref[...], preferred_element_type=jnp.float32) \
        + bp2_ref[...].reshape(1, D)
    deltaA = jnp.dot(h, W2A_ref[...], preferred_element_type=jnp.float32) \
        + b2A_ref[...].reshape(1, D)

    qA = qA_ref[0]
    qA_rep = jnp.broadcast_to(qA.reshape(M, 1, D), (M, K, D)).reshape(M * K, D)
    z = jnp.maximum(qA_rep - g[:, 0:D] + deltaA, 0.0)
    a = jnp.dot(z, Wa2_ref[...], preferred_element_type=jnp.float32) \
        + ba2_ref[...].reshape(1, D)

    a3 = a.reshape(M, K, D)
    amax = jnp.max(a3, axis=1, keepdims=True)
    e = jnp.exp(a3 - amax)
    w = e / jnp.sum(e, axis=1, keepdims=True)

    vd = (g[:, D:2 * D] + delta).reshape(M, K, D)
    out_ref[0] = jnp.sum(w * vd, axis=1)


def _fused_attention(qA, aP, g, Wp2, bp2, W2A, b2A, Wa2, ba2):
    B, N, D = qA.shape
    K = g.shape[2]
    M = 128
    grid = (B, N // M)
    wspec = lambda shape: pl.BlockSpec(shape, lambda b, i: (0,) * len(shape))
    return pl.pallas_call(
        _attn_body,
        grid=grid,
        in_specs=[
            pl.BlockSpec((1, M, D), lambda b, i: (b, i, 0)),
            pl.BlockSpec((1, M, D), lambda b, i: (b, i, 0)),
            pl.BlockSpec((1, M, K, 3 * D), lambda b, i: (b, i, 0, 0)),
            wspec((D, D)), wspec((D,)),
            wspec((D, D)), wspec((D,)),
            wspec((D, D)), wspec((D,)),
        ],
        out_specs=pl.BlockSpec((1, M, D), lambda b, i: (b, i, 0)),
        out_shape=jax.ShapeDtypeStruct((B, N, D), jnp.float32),
    )(qA, aP, g, Wp2, bp2, W2A, b2A, Wa2, ba2)


def kernel(xyz, features, Wq, bq, Wk, bk, Wv, bv, Wp1, bp1, Wp2, bp2,
           Wa1, ba1, Wa2, ba2):
    B, N, D = features.shape

    sq = jnp.sum(xyz * xyz, axis=-1, keepdims=True)
    pad5 = jnp.zeros(xyz.shape[:2] + (5,), xyz.dtype)
    xt = jnp.concatenate([xyz, pad5], axis=-1)
    idx = jnp.transpose(
        _knn_topk(xt, jnp.transpose(xt, (0, 2, 1)), sq,
                  jnp.transpose(sq, (0, 2, 1))),
        (0, 2, 1))

    WqA = Wq @ Wa1
    WkA = Wk @ Wa1
    W2A = Wp2 @ Wa1
    qA = features @ WqA + (bq @ Wa1)
    kA = features @ WkA + (bk @ Wa1)
    v = features @ Wv + bv
    b2A = bp2 @ Wa1 + ba1

    K = K_NN
    bP = xyz @ Wp1
    aP = bP + bp1
    table = jnp.concatenate([kA, v, bP], axis=-1).reshape(B * N, 3 * D)
    gidx = (idx + (jnp.arange(B, dtype=jnp.int32) * N)[:, None, None])
    g = _sc_gather(table, gidx.reshape(-1)).reshape(B, N, K, 3 * D)

    return _fused_attention(qA, aP, g, Wp2, bp2, W2A, b2A, Wa2, ba2)


# speedup vs baseline: 1.0995x; 1.0995x over previous
import functools

import jax
import jax.numpy as jnp
from jax import lax
from jax.experimental import pallas as pl
from jax.experimental.pallas import tpu as pltpu
from jax.experimental.pallas import tpu_sc as plsc

K_NN = 16


def _knn_body(xt_ref, ytT_ref, sx_ref, sy_ref, idx_ref, dist_ref):
    M = xt_ref.shape[1]
    N = ytT_ref.shape[2]
    K = idx_ref.shape[1]
    c = jnp.dot(xt_ref[0], ytT_ref[0], preferred_element_type=jnp.float32)
    d = (-2.0 * c + sx_ref[0]) + sy_ref[0]
    dist_ref[...] = d
    col = jax.lax.broadcasted_iota(jnp.int32, (M, N), 1)

    def step(it, _):
        dd = dist_ref[...]
        m = jnp.min(dd, axis=1, keepdims=True)
        am = jnp.min(jnp.where(dd == m, col, N), axis=1, keepdims=True)
        dist_ref[...] = jnp.where(col == am, jnp.inf, dd)
        idx_ref[0, pl.ds(it, 1), :] = am.reshape(1, M)
        return 0

    jax.lax.fori_loop(0, K, step, 0, unroll=False)


def _knn_topk(xt, ytT, sx, sy):
    B, N, _ = xt.shape
    M = 256
    return pl.pallas_call(
        _knn_body,
        grid=(B, N // M),
        in_specs=[
            pl.BlockSpec((1, M, 8), lambda b, i: (b, i, 0)),
            pl.BlockSpec((1, 8, N), lambda b, i: (b, 0, 0)),
            pl.BlockSpec((1, M, 1), lambda b, i: (b, i, 0)),
            pl.BlockSpec((1, 1, N), lambda b, i: (b, 0, 0)),
        ],
        out_specs=pl.BlockSpec((1, K_NN, M), lambda b, i: (b, 0, i)),
        out_shape=jax.ShapeDtypeStruct((B, K_NN, N), jnp.int32),
        scratch_shapes=[pltpu.VMEM((M, N), jnp.float32)],
    )(xt, ytT, sx, sy)


def _sc_gather(table, gidx):
    R, W = table.shape
    P = gidx.shape[0]
    info = plsc.get_sparse_core_info()
    NC, NS = info.num_cores, info.num_subcores
    NW = NC * NS
    C = 128
    per_w = P // NW
    n_chunks = per_w // C
    mesh = plsc.VectorSubcoreMesh(core_axis_name="c", subcore_axis_name="s")

    @functools.partial(
        pl.kernel, mesh=mesh,
        out_type=jax.ShapeDtypeStruct((P, W), jnp.float32),
        scratch_types=[
            pltpu.VMEM((C,), jnp.int32),
            pltpu.VMEM((C, W), jnp.float32),
            pltpu.SemaphoreType.DMA,
        ],
    )
    def k(table_h, gidx_h, out_h, idxv, buf, sem):
        wid = lax.axis_index("s") * NC + lax.axis_index("c")
        base = wid * per_w

        def body(j, _):
            off = base + j * C
            pltpu.sync_copy(gidx_h.at[pl.ds(off, C)], idxv)
            pltpu.async_copy(table_h.at[idxv], buf, sem).wait()
            pltpu.sync_copy(buf, out_h.at[pl.ds(off, C)])
            return 0

        lax.fori_loop(0, n_chunks, body, 0)

    return k(table, gidx)


def _attn_body(qA_ref, aP_ref, g_ref,
               Wp2_ref, bp2_ref, W2A_ref, b2A_ref,
               Wa2_ref, ba2_ref, out_ref):
    M = qA_ref.shape[1]
    K = g_ref.shape[2]
    D = qA_ref.shape[2]

    g = g_ref[0].reshape(M * K, 3 * D)
    aP = aP_ref[0]
    aP_rep = jnp.broadcast_to(aP.reshape(M, 1, D), (M, K, D)).reshape(M * K, D)
    h = jnp.maximum(aP_rep - g[:, 2 * D:3 * D], 0.0)

    delta = jnp.dot(h, Wp2_ref[...], preferred_element_type=jnp.float32) \
        + bp2_ref[...].reshape(1, D)
    deltaA = jnp.dot(h, W2A_ref[...], preferred_element_type=jnp.float32) \
        + b2A_ref[...].reshape(1, D)

    qA = qA_ref[0]
    qA_rep = jnp.broadcast_to(qA.reshape(M, 1, D), (M, K, D)).reshape(M * K, D)
    z = jnp.maximum(qA_rep - g[:, 0:D] + deltaA, 0.0)
    a = jnp.dot(z, Wa2_ref[...], preferred_element_type=jnp.float32) \
        + ba2_ref[...].reshape(1, D)

    a3 = a.reshape(M, K, D)
    amax = jnp.max(a3, axis=1, keepdims=True)
    e = jnp.exp(a3 - amax)
    w = e / jnp.sum(e, axis=1, keepdims=True)

    vd = (g[:, D:2 * D] + delta).reshape(M, K, D)
    out_ref[0] = jnp.sum(w * vd, axis=1)


def _fused_attention(qA, aP, g, Wp2, bp2, W2A, b2A, Wa2, ba2):
    B, N, D = qA.shape
    K = g.shape[2]
    M = 128
    grid = (B, N // M)
    wspec = lambda shape: pl.BlockSpec(shape, lambda b, i: (0,) * len(shape))
    return pl.pallas_call(
        _attn_body,
        grid=grid,
        in_specs=[
            pl.BlockSpec((1, M, D), lambda b, i: (b, i, 0)),
            pl.BlockSpec((1, M, D), lambda b, i: (b, i, 0)),
            pl.BlockSpec((1, M, K, 3 * D), lambda b, i: (b, i, 0, 0)),
            wspec((D, D)), wspec((D,)),
            wspec((D, D)), wspec((D,)),
            wspec((D, D)), wspec((D,)),
        ],
        out_specs=pl.BlockSpec((1, M, D), lambda b, i: (b, i, 0)),
        out_shape=jax.ShapeDtypeStruct((B, N, D), jnp.float32),
    )(qA, aP, g, Wp2, bp2, W2A, b2A, Wa2, ba2)


def kernel(xyz, features, Wq, bq, Wk, bk, Wv, bv, Wp1, bp1, Wp2, bp2,
           Wa1, ba1, Wa2, ba2):
    B, N, D = features.shape

    sq = jnp.sum(xyz * xyz, axis=-1, keepdims=True)
    pad5 = jnp.zeros(xyz.shape[:2] + (5,), xyz.dtype)
    xt = jnp.concatenate([xyz, pad5], axis=-1)
    idx = jnp.transpose(
        _knn_topk(xt, jnp.transpose(xt, (0, 2, 1)), sq,
                  jnp.transpose(sq, (0, 2, 1))),
        (0, 2, 1))

    WqA = Wq @ Wa1
    WkA = Wk @ Wa1
    W2A = Wp2 @ Wa1
    qA = features @ WqA + (bq @ Wa1)
    kA = features @ WkA + (bk @ Wa1)
    v = features @ Wv + bv
    b2A = bp2 @ Wa1 + ba1

    K = K_NN
    bP = xyz @ Wp1
    aP = bP + bp1
    table = jnp.concatenate([kA, v, bP], axis=-1).reshape(B * N, 3 * D)
    gidx = (idx + (jnp.arange(B, dtype=jnp.int32) * N)[:, None, None])
    g = _sc_gather(table, gidx.reshape(-1)).reshape(B, N, K, 3 * D)

    return _fused_attention(qA, aP, g, Wp2, bp2, W2A, b2A, Wa2, ba2)


# speedup vs baseline: 1.1640x; 1.0587x over previous
import functools

import jax
import jax.numpy as jnp
from jax import lax
from jax.experimental import pallas as pl
from jax.experimental.pallas import tpu as pltpu
from jax.experimental.pallas import tpu_sc as plsc

K_NN = 16


def _knn_body(xt_ref, ytT_ref, sx_ref, sy_ref, idx_ref, dist_ref):
    M = xt_ref.shape[1]
    N = ytT_ref.shape[2]
    K = idx_ref.shape[1]
    c = jnp.dot(xt_ref[0], ytT_ref[0], preferred_element_type=jnp.float32)
    d = (-2.0 * c + sx_ref[0]) + sy_ref[0]
    dist_ref[...] = d
    col = jax.lax.broadcasted_iota(jnp.int32, (M, N), 1)

    def step(it, _):
        dd = dist_ref[...]
        m = jnp.min(dd, axis=1, keepdims=True)
        am = jnp.min(jnp.where(dd == m, col, N), axis=1, keepdims=True)
        dist_ref[...] = jnp.where(col == am, jnp.inf, dd)
        idx_ref[0, pl.ds(it, 1), :] = am.reshape(1, M)
        return 0

    jax.lax.fori_loop(0, K, step, 0, unroll=False)


def _knn_topk(xt, ytT, sx, sy):
    B, N, _ = xt.shape
    M = 256
    return pl.pallas_call(
        _knn_body,
        grid=(B, N // M),
        in_specs=[
            pl.BlockSpec((1, M, 8), lambda b, i: (b, i, 0)),
            pl.BlockSpec((1, 8, N), lambda b, i: (b, 0, 0)),
            pl.BlockSpec((1, M, 1), lambda b, i: (b, i, 0)),
            pl.BlockSpec((1, 1, N), lambda b, i: (b, 0, 0)),
        ],
        out_specs=pl.BlockSpec((1, K_NN, M), lambda b, i: (b, 0, i)),
        out_shape=jax.ShapeDtypeStruct((B, K_NN, N), jnp.int32),
        scratch_shapes=[pltpu.VMEM((M, N), jnp.float32)],
    )(xt, ytT, sx, sy)


def _sc_gather(table, gidx):
    R, W = table.shape
    P = gidx.shape[0]
    info = plsc.get_sparse_core_info()
    NC, NS = info.num_cores, info.num_subcores
    NW = NC * NS
    C = 128
    per_w = P // NW
    n_chunks = per_w // C
    mesh = plsc.VectorSubcoreMesh(core_axis_name="c", subcore_axis_name="s")

    @functools.partial(
        pl.kernel, mesh=mesh,
        out_type=jax.ShapeDtypeStruct((P, W), jnp.int32),
        scratch_types=[
            pltpu.VMEM((C,), jnp.int32),
            pltpu.VMEM((C, W), jnp.int32),
            pltpu.SemaphoreType.DMA,
        ],
    )
    def k(table_h, gidx_h, out_h, idxv, buf, sem):
        wid = lax.axis_index("s") * NC + lax.axis_index("c")
        base = wid * per_w

        def body(j, _):
            off = base + j * C
            pltpu.sync_copy(gidx_h.at[pl.ds(off, C)], idxv)
            pltpu.async_copy(table_h.at[idxv], buf, sem).wait()
            pltpu.sync_copy(buf, out_h.at[pl.ds(off, C)])
            return 0

        lax.fori_loop(0, n_chunks, body, 0)

    return k(table, gidx)


def _attn_body(qA_ref, aP_ref, g_ref,
               Wp2_ref, bp2_ref, W2A_ref, b2A_ref,
               Wa2_ref, ba2_ref, out_ref):
    M = qA_ref.shape[1]
    K = g_ref.shape[2]
    D = qA_ref.shape[2]

    gi = g_ref[0].reshape(M * K, 2 * D)
    kv = gi[:, 0:D]
    kAg = jax.lax.bitcast_convert_type(kv & jnp.int32(-0x10000), jnp.float32)
    vg = jax.lax.bitcast_convert_type(jax.lax.shift_left(kv, 16),
                                      jnp.float32)
    bg = jax.lax.bitcast_convert_type(gi[:, D:2 * D], jnp.float32)
    aP = aP_ref[0]
    aP_rep = jnp.broadcast_to(aP.reshape(M, 1, D), (M, K, D)).reshape(M * K, D)
    h = jnp.maximum(aP_rep - bg, 0.0)

    delta = jnp.dot(h, Wp2_ref[...], preferred_element_type=jnp.float32) \
        + bp2_ref[...].reshape(1, D)
    deltaA = jnp.dot(h, W2A_ref[...], preferred_element_type=jnp.float32) \
        + b2A_ref[...].reshape(1, D)

    qA = qA_ref[0]
    qA_rep = jnp.broadcast_to(qA.reshape(M, 1, D), (M, K, D)).reshape(M * K, D)
    z = jnp.maximum(qA_rep - kAg + deltaA, 0.0)
    a = jnp.dot(z, Wa2_ref[...], preferred_element_type=jnp.float32) \
        + ba2_ref[...].reshape(1, D)

    a3 = a.reshape(M, K, D)
    amax = jnp.max(a3, axis=1, keepdims=True)
    e = jnp.exp(a3 - amax)
    w = e / jnp.sum(e, axis=1, keepdims=True)

    vd = (vg + delta).reshape(M, K, D)
    out_ref[0] = jnp.sum(w * vd, axis=1)


def _fused_attention(qA, aP, g, Wp2, bp2, W2A, b2A, Wa2, ba2):
    B, N, D = qA.shape
    K = g.shape[2]
    M = 128
    grid = (B, N // M)
    wspec = lambda shape: pl.BlockSpec(shape, lambda b, i: (0,) * len(shape))
    return pl.pallas_call(
        _attn_body,
        grid=grid,
        in_specs=[
            pl.BlockSpec((1, M, D), lambda b, i: (b, i, 0)),
            pl.BlockSpec((1, M, D), lambda b, i: (b, i, 0)),
            pl.BlockSpec((1, M, K, 2 * D), lambda b, i: (b, i, 0, 0)),
            wspec((D, D)), wspec((D,)),
            wspec((D, D)), wspec((D,)),
            wspec((D, D)), wspec((D,)),
        ],
        out_specs=pl.BlockSpec((1, M, D), lambda b, i: (b, i, 0)),
        out_shape=jax.ShapeDtypeStruct((B, N, D), jnp.float32),
    )(qA, aP, g, Wp2, bp2, W2A, b2A, Wa2, ba2)


def kernel(xyz, features, Wq, bq, Wk, bk, Wv, bv, Wp1, bp1, Wp2, bp2,
           Wa1, ba1, Wa2, ba2):
    B, N, D = features.shape

    sq = jnp.sum(xyz * xyz, axis=-1, keepdims=True)
    pad5 = jnp.zeros(xyz.shape[:2] + (5,), xyz.dtype)
    xt = jnp.concatenate([xyz, pad5], axis=-1)
    idx = jnp.transpose(
        _knn_topk(xt, jnp.transpose(xt, (0, 2, 1)), sq,
                  jnp.transpose(sq, (0, 2, 1))),
        (0, 2, 1))

    WqA = Wq @ Wa1
    WkA = Wk @ Wa1
    W2A = Wp2 @ Wa1
    qA = features @ WqA + (bq @ Wa1)
    kA = features @ WkA + (bk @ Wa1)
    v = features @ Wv + bv
    b2A = bp2 @ Wa1 + ba1

    K = K_NN
    bP = xyz @ Wp1
    aP = bP + bp1
    kv = ((jax.lax.bitcast_convert_type(kA, jnp.int32) + 0x8000)
          & (-0x10000)) | \
        jax.lax.shift_right_logical(
            jax.lax.bitcast_convert_type(v, jnp.int32) + 0x8000, 16)
    table = jnp.concatenate(
        [kv, jax.lax.bitcast_convert_type(bP, jnp.int32)],
        axis=-1).reshape(B * N, 2 * D)
    gidx = (idx + (jnp.arange(B, dtype=jnp.int32) * N)[:, None, None])
    g = _sc_gather(table, gidx.reshape(-1)).reshape(B, N, K, 2 * D)

    return _fused_attention(qA, aP, g, Wp2, bp2, W2A, b2A, Wa2, ba2)
